# baseline (speedup 1.0000x reference)
import functools

import jax
import jax.numpy as jnp
from jax import lax
from jax.experimental import pallas as pl
from jax.experimental.pallas import tpu as pltpu
from jax.experimental.pallas import tpu_sc as plsc

F = 26
B = 4096
D = 32
V = 1000000

NC = 2
NS = 16
NW = NC * NS

CHUNK = B // NW

VBLK = 8192
GRID = -(-V // VBLK)
NROWS = GRID * VBLK * D // 128


def _transpose_body(in_ref, out_ref):
  x = in_ref[...].reshape(D, VBLK)
  for c in range(VBLK // 512):
    xs = jnp.concatenate(
        [x[:, 512 * c + 128 * a:512 * c + 128 * (a + 1)] for a in range(4)],
        axis=0,
    )
    out_ref[128 * c:128 * (c + 1), :] = xs.T


_TRANSPOSE = pl.pallas_call(
    _transpose_body,
    grid=(GRID,),
    in_specs=[pl.BlockSpec((4, 8, VBLK), lambda j: (0, 0, j))],
    out_specs=pl.BlockSpec((VBLK * D // 128, 128), lambda j: (j, 0)),
    out_shape=jax.ShapeDtypeStruct((NROWS, 128), jnp.float32),
)


def _permuted_rows(values):
  v = values
  return (
      (v & ~(VBLK - 1))
      + ((v >> 9) & (VBLK // 512 - 1)) * 512
      + ((v & 127) << 2)
      + ((v >> 7) & 3)
  )


def _regroup_body(in_ref, out_ref):
  w = in_ref[...]
  wt = w.T
  for q in range(4):
    out_ref[0, :, 128 * q:128 * (q + 1)] = wt[D * q:D * (q + 1), :]


_REGROUP = pl.pallas_call(
    _regroup_body,
    grid=(F * 8,),
    in_specs=[pl.BlockSpec((128, 128), lambda j: (j, 0))],
    out_specs=pl.BlockSpec((1, D, 512), lambda j: (j // 8, 0, j % 8)),
    out_shape=jax.ShapeDtypeStruct((F, D, B), jnp.float32),
)


def _build_gather():
  mesh = plsc.VectorSubcoreMesh(core_axis_name="c", subcore_axis_name="s")

  @functools.partial(
      pl.kernel,
      out_type=jax.ShapeDtypeStruct((F, 8, CHUNK, 4 * D), jnp.float32),
      mesh=mesh,
      scratch_types=[
          pltpu.VMEM((F, CHUNK), jnp.int32),
          pltpu.VMEM((F, CHUNK, D), jnp.float32),
          pltpu.SemaphoreType.DMA,
      ],
      compiler_params=pltpu.CompilerParams(use_tc_tiling_on_sc=False),
  )
  def gather_kernel(tab_hbm, idx_hbm, out_hbm, idx_v, rows_v, sem):
    wid = lax.axis_index("s") * NC + lax.axis_index("c")
    bc = wid // 4
    q = wid % 4
    pltpu.sync_copy(idx_hbm.at[wid], idx_v)
    copies = [
        pltpu.async_copy(tab_hbm.at[idx_v.at[f]], rows_v.at[f], sem)
        for f in range(F)
    ]
    for c in copies:
      c.wait()
    for f in range(F):
      pltpu.sync_copy(
          rows_v.at[f], out_hbm.at[f, bc, :, pl.ds(q * D, D)]
      )

  return gather_kernel


_GATHER = _build_gather()


def kernel(table, values, lengths):
  tab3 = table.T.reshape(4, 8, V)
  tablin = _TRANSPOSE(tab3)
  tab_flat = tablin.reshape(GRID * VBLK, D)
  idx = _permuted_rows(values).reshape(F, NW, CHUNK).transpose(1, 0, 2)
  out5 = _GATHER(tab_flat, idx)
  emb = _REGROUP(out5.reshape(F * 8 * CHUNK, 4 * D))
  split_embeddings = emb.transpose(0, 2, 1)
  split_lengths = lengths.reshape(F, B)
  reduce_lengths = split_lengths.sum(axis=1)
  offsets = jnp.concatenate([
      jnp.zeros((1,), dtype=reduce_lengths.dtype),
      jnp.cumsum(reduce_lengths),
  ])
  return split_embeddings, split_lengths, offsets


# speedup vs baseline: 1.2959x; 1.2959x over previous
import functools

import jax
import jax.numpy as jnp
from jax import lax
from jax.experimental import pallas as pl
from jax.experimental.pallas import tpu as pltpu
from jax.experimental.pallas import tpu_sc as plsc

F = 26
B = 4096
D = 32
V = 1000000

NC = 2
NS = 16
NW = NC * NS

CHUNK = 128

VBLK = 8192
GRID = -(-V // VBLK)
NROWS = GRID * VBLK * D // 128


def _transpose_body(in_ref, out_ref):
  x = in_ref[...].reshape(D, VBLK)
  for c in range(VBLK // 512):
    xs = jnp.concatenate(
        [x[:, 512 * c + 128 * a:512 * c + 128 * (a + 1)] for a in range(4)],
        axis=0,
    )
    out_ref[128 * c:128 * (c + 1), :] = xs.T


_TRANSPOSE = pl.pallas_call(
    _transpose_body,
    grid=(GRID,),
    in_specs=[pl.BlockSpec((4, 8, VBLK), lambda j: (0, 0, j))],
    out_specs=pl.BlockSpec((VBLK * D // 128, 128), lambda j: (j, 0)),
    out_shape=jax.ShapeDtypeStruct((NROWS, 128), jnp.float32),
)


def _permuted_rows(values):
  v = values
  return (
      (v & ~(VBLK - 1))
      + ((v >> 9) & (VBLK // 512 - 1)) * 512
      + ((v & 127) << 2)
      + ((v >> 7) & 3)
  )


def _build_gather(total, d):
  per_w = total // NW
  n_chunks = per_w // CHUNK

  mesh = plsc.VectorSubcoreMesh(core_axis_name="c", subcore_axis_name="s")

  @functools.partial(
      pl.kernel,
      out_type=jax.ShapeDtypeStruct((total, d), jnp.float32),
      mesh=mesh,
      scratch_types=[
          pltpu.VMEM((n_chunks, CHUNK), jnp.int32),
          pltpu.VMEM((per_w, d), jnp.float32),
          pltpu.SemaphoreType.DMA,
      ],
      compiler_params=pltpu.CompilerParams(use_tc_tiling_on_sc=False),
  )
  def gather_kernel(table_hbm, idx_hbm, out_hbm, idx_v, rows_v, sem):
    wid = lax.axis_index("s") * NC + lax.axis_index("c")
    base = wid * per_w
    pltpu.sync_copy(idx_hbm.at[wid], idx_v)
    copies = []
    for j in range(n_chunks):
      copies.append(
          pltpu.async_copy(
              table_hbm.at[idx_v.at[j]],
              rows_v.at[pl.ds(j * CHUNK, CHUNK)],
              sem,
          )
      )
    for c in copies:
      c.wait()
    pltpu.sync_copy(rows_v, out_hbm.at[pl.ds(base, per_w)])

  return gather_kernel


_GATHER = _build_gather(F * B, D)


def kernel(table, values, lengths):
  tab3 = table.T.reshape(4, 8, V)
  tablin = _TRANSPOSE(tab3)
  tab_flat = tablin.reshape(GRID * VBLK, D)
  idx = _permuted_rows(values).reshape(NW, (F * B) // NW // CHUNK, CHUNK)
  rows = _GATHER(tab_flat, idx)
  split_embeddings = rows.reshape(F, B, D)
  split_lengths = lengths.reshape(F, B)
  reduce_lengths = split_lengths.sum(axis=1)
  offsets = jnp.concatenate([
      jnp.zeros((1,), dtype=reduce_lengths.dtype),
      jnp.cumsum(reduce_lengths),
  ])
  return split_embeddings, split_lengths, offsets


# speedup vs baseline: 1.5476x; 1.1942x over previous
import functools

import jax
import jax.numpy as jnp
from jax import lax
from jax.experimental import pallas as pl
from jax.experimental.pallas import tpu as pltpu
from jax.experimental.pallas import tpu_sc as plsc

F = 26
B = 4096
D = 32
V = 1000000

NC = 2
NS = 16
NW = NC * NS

CHUNK = 128

VBLK = 16384
GRID = -(-V // VBLK)
NROWS = GRID * VBLK * D // 128


def _transpose_body(in_ref, out_ref):
  x = in_ref[...].reshape(D, VBLK)
  for c in range(VBLK // 512):
    xs = jnp.concatenate(
        [x[:, 512 * c + 128 * a:512 * c + 128 * (a + 1)] for a in range(4)],
        axis=0,
    )
    out_ref[128 * c:128 * (c + 1), :] = xs.T


_TRANSPOSE = pl.pallas_call(
    _transpose_body,
    grid=(GRID,),
    in_specs=[pl.BlockSpec((4, 8, VBLK), lambda j: (0, 0, j))],
    out_specs=pl.BlockSpec((VBLK * D // 128, 128), lambda j: (j, 0)),
    out_shape=jax.ShapeDtypeStruct((NROWS, 128), jnp.float32),
)


def _permuted_rows(values):
  v = values
  return (
      (v & ~(VBLK - 1))
      + ((v >> 9) & (VBLK // 512 - 1)) * 512
      + ((v & 127) << 2)
      + ((v >> 7) & 3)
  )


def _build_gather(total, d):
  per_w = total // NW
  n_chunks = per_w // CHUNK

  mesh = plsc.VectorSubcoreMesh(core_axis_name="c", subcore_axis_name="s")

  @functools.partial(
      pl.kernel,
      out_type=jax.ShapeDtypeStruct((total, d), jnp.float32),
      mesh=mesh,
      scratch_types=[
          pltpu.VMEM((n_chunks, CHUNK), jnp.int32),
          pltpu.VMEM((per_w, d), jnp.float32),
          pltpu.SemaphoreType.DMA,
      ],
      compiler_params=pltpu.CompilerParams(use_tc_tiling_on_sc=False),
  )
  def gather_kernel(table_hbm, idx_hbm, out_hbm, idx_v, rows_v, sem):
    wid = lax.axis_index("s") * NC + lax.axis_index("c")
    base = wid * per_w
    pltpu.sync_copy(idx_hbm.at[wid], idx_v)
    copies = []
    for j in range(n_chunks):
      copies.append(
          pltpu.async_copy(
              table_hbm.at[idx_v.at[j]],
              rows_v.at[pl.ds(j * CHUNK, CHUNK)],
              sem,
          )
      )
    for c in copies:
      c.wait()
    pltpu.sync_copy(rows_v, out_hbm.at[pl.ds(base, per_w)])

  return gather_kernel


_GATHER = _build_gather(F * B, D)


def kernel(table, values, lengths):
  tab3 = table.T.reshape(4, 8, V)
  tablin = _TRANSPOSE(tab3)
  tab_flat = tablin.reshape(GRID * VBLK, D)
  idx = _permuted_rows(values).reshape(NW, (F * B) // NW // CHUNK, CHUNK)
  rows = _GATHER(tab_flat, idx)
  split_embeddings = rows.reshape(F, B, D)
  split_lengths = lengths.reshape(F, B)
  reduce_lengths = split_lengths.sum(axis=1)
  offsets = jnp.concatenate([
      jnp.zeros((1,), dtype=reduce_lengths.dtype),
      jnp.cumsum(reduce_lengths),
  ])
  return split_embeddings, split_lengths, offsets


# speedup vs baseline: 1.6923x; 1.0935x over previous
import functools

import jax
import jax.numpy as jnp
from jax import lax
from jax.experimental import pallas as pl
from jax.experimental.pallas import tpu as pltpu
from jax.experimental.pallas import tpu_sc as plsc

F = 26
B = 4096
D = 32
V = 1000000

NC = 2
NS = 16
NW = NC * NS

CHUNK = 128

VBLK = 32768
GRID = -(-V // VBLK)
NROWS = GRID * VBLK * D // 128


def _transpose_body(in_ref, out_ref):
  x = in_ref[...].reshape(D, VBLK)
  for c in range(VBLK // 512):
    xs = jnp.concatenate(
        [x[:, 512 * c + 128 * a:512 * c + 128 * (a + 1)] for a in range(4)],
        axis=0,
    )
    out_ref[128 * c:128 * (c + 1), :] = xs.T


_TRANSPOSE = pl.pallas_call(
    _transpose_body,
    grid=(GRID,),
    in_specs=[pl.BlockSpec((4, 8, VBLK), lambda j: (0, 0, j))],
    out_specs=pl.BlockSpec((VBLK * D // 128, 128), lambda j: (j, 0)),
    out_shape=jax.ShapeDtypeStruct((NROWS, 128), jnp.float32),
)


def _permuted_rows(values):
  v = values
  return (
      (v & ~(VBLK - 1))
      + ((v >> 9) & (VBLK // 512 - 1)) * 512
      + ((v & 127) << 2)
      + ((v >> 7) & 3)
  )


def _build_gather(total, d):
  per_w = total // NW
  n_chunks = per_w // CHUNK

  mesh = plsc.VectorSubcoreMesh(core_axis_name="c", subcore_axis_name="s")

  @functools.partial(
      pl.kernel,
      out_type=jax.ShapeDtypeStruct((total, d), jnp.float32),
      mesh=mesh,
      scratch_types=[
          pltpu.VMEM((n_chunks, CHUNK), jnp.int32),
          pltpu.VMEM((per_w, d), jnp.float32),
          pltpu.SemaphoreType.DMA,
      ],
      compiler_params=pltpu.CompilerParams(use_tc_tiling_on_sc=False),
  )
  def gather_kernel(table_hbm, idx_hbm, out_hbm, idx_v, rows_v, sem):
    wid = lax.axis_index("s") * NC + lax.axis_index("c")
    base = wid * per_w
    pltpu.sync_copy(idx_hbm.at[wid], idx_v)
    copies = []
    for j in range(n_chunks):
      copies.append(
          pltpu.async_copy(
              table_hbm.at[idx_v.at[j]],
              rows_v.at[pl.ds(j * CHUNK, CHUNK)],
              sem,
          )
      )
    for c in copies:
      c.wait()
    pltpu.sync_copy(rows_v, out_hbm.at[pl.ds(base, per_w)])

  return gather_kernel


_GATHER = _build_gather(F * B, D)


def kernel(table, values, lengths):
  tab3 = table.T.reshape(4, 8, V)
  tablin = _TRANSPOSE(tab3)
  tab_flat = tablin.reshape(GRID * VBLK, D)
  idx = _permuted_rows(values).reshape(NW, (F * B) // NW // CHUNK, CHUNK)
  rows = _GATHER(tab_flat, idx)
  split_embeddings = rows.reshape(F, B, D)
  split_lengths = lengths.reshape(F, B)
  reduce_lengths = split_lengths.sum(axis=1)
  offsets = jnp.concatenate([
      jnp.zeros((1,), dtype=reduce_lengths.dtype),
      jnp.cumsum(reduce_lengths),
  ])
  return split_embeddings, split_lengths, offsets


# speedup vs baseline: 1.7037x; 1.0067x over previous
import functools

import jax
import jax.numpy as jnp
from jax import lax
from jax.experimental import pallas as pl
from jax.experimental.pallas import tpu as pltpu
from jax.experimental.pallas import tpu_sc as plsc

F = 26
B = 4096
D = 32
V = 1000000

NC = 2
NS = 16
NW = NC * NS

CHUNK = 128

VBLK = 65536
GRID = -(-V // VBLK)
NROWS = GRID * VBLK * D // 128


def _transpose_body(in_ref, out_ref):
  x = in_ref[...].reshape(D, VBLK)
  for c in range(VBLK // 512):
    xs = jnp.concatenate(
        [x[:, 512 * c + 128 * a:512 * c + 128 * (a + 1)] for a in range(4)],
        axis=0,
    )
    out_ref[128 * c:128 * (c + 1), :] = xs.T


_TRANSPOSE = pl.pallas_call(
    _transpose_body,
    grid=(GRID,),
    in_specs=[pl.BlockSpec((4, 8, VBLK), lambda j: (0, 0, j))],
    out_specs=pl.BlockSpec((VBLK * D // 128, 128), lambda j: (j, 0)),
    out_shape=jax.ShapeDtypeStruct((NROWS, 128), jnp.float32),
)


def _permuted_rows(values):
  v = values
  return (
      (v & ~(VBLK - 1))
      + ((v >> 9) & (VBLK // 512 - 1)) * 512
      + ((v & 127) << 2)
      + ((v >> 7) & 3)
  )


def _build_gather(total, d):
  per_w = total // NW
  n_chunks = per_w // CHUNK

  mesh = plsc.VectorSubcoreMesh(core_axis_name="c", subcore_axis_name="s")

  @functools.partial(
      pl.kernel,
      out_type=jax.ShapeDtypeStruct((total, d), jnp.float32),
      mesh=mesh,
      scratch_types=[
          pltpu.VMEM((n_chunks, CHUNK), jnp.int32),
          pltpu.VMEM((per_w, d), jnp.float32),
          pltpu.SemaphoreType.DMA,
      ],
      compiler_params=pltpu.CompilerParams(use_tc_tiling_on_sc=False),
  )
  def gather_kernel(table_hbm, idx_hbm, out_hbm, idx_v, rows_v, sem):
    wid = lax.axis_index("s") * NC + lax.axis_index("c")
    base = wid * per_w
    pltpu.sync_copy(idx_hbm.at[wid], idx_v)
    copies = []
    for j in range(n_chunks):
      copies.append(
          pltpu.async_copy(
              table_hbm.at[idx_v.at[j]],
              rows_v.at[pl.ds(j * CHUNK, CHUNK)],
              sem,
          )
      )
    for c in copies:
      c.wait()
    pltpu.sync_copy(rows_v, out_hbm.at[pl.ds(base, per_w)])

  return gather_kernel


_GATHER = _build_gather(F * B, D)


def kernel(table, values, lengths):
  tab3 = table.T.reshape(4, 8, V)
  tablin = _TRANSPOSE(tab3)
  tab_flat = tablin.reshape(GRID * VBLK, D)
  idx = _permuted_rows(values).reshape(NW, (F * B) // NW // CHUNK, CHUNK)
  rows = _GATHER(tab_flat, idx)
  split_embeddings = rows.reshape(F, B, D)
  split_lengths = lengths.reshape(F, B)
  reduce_lengths = split_lengths.sum(axis=1)
  offsets = jnp.concatenate([
      jnp.zeros((1,), dtype=reduce_lengths.dtype),
      jnp.cumsum(reduce_lengths),
  ])
  return split_embeddings, split_lengths, offsets
